# Initial kernel scaffold; baseline (speedup 1.0000x reference)
import jax
import jax.numpy as jnp
from jax.experimental import pallas as pl


def kernel(item_indices, Pu, Eu, Item, W1, b1, W2, b2, W3, b3, Wo, bo):
    raise NotImplementedError("write your pallas kernel here")


# speedup vs baseline: 1.8071x; 1.8071x over previous
import functools

import jax
import jax.numpy as jnp
from jax import lax
from jax.experimental import pallas as pl
from jax.experimental.pallas import tpu as pltpu
from jax.experimental.pallas import tpu_sc as plsc


def _make_sc_gather(V, D, B):
    info = plsc.get_sparse_core_info()
    NC, NS = info.num_cores, info.num_subcores
    NW = NC * NS
    b_per_w = B // NW
    assert B % NW == 0 and b_per_w % 128 == 0
    nchunk = b_per_w // 128
    mesh = plsc.VectorSubcoreMesh(core_axis_name="c", subcore_axis_name="s")

    @functools.partial(
        pl.kernel,
        mesh=mesh,
        out_type=jax.ShapeDtypeStruct((B, D), jnp.float32),
        scratch_types=[
            pltpu.VMEM((nchunk, 128), jnp.int32),
            pltpu.VMEM((b_per_w, D), jnp.float32),
            pltpu.SemaphoreType.DMA,
        ],
    )
    def gather_k(idx_hbm, table_hbm, out_hbm, idx_v, rows_v, sem):
        wid = lax.axis_index("s") * NC + lax.axis_index("c")
        base = wid * b_per_w
        pltpu.sync_copy(idx_hbm.at[wid], idx_v)
        copies = [
            pltpu.async_copy(
                table_hbm.at[idx_v.at[j]],
                rows_v.at[pl.ds(j * 128, 128)],
                sem,
            )
            for j in range(nchunk)
        ]
        for c in copies:
            c.wait()
        pltpu.sync_copy(rows_v, out_hbm.at[pl.ds(base, b_per_w)])

    def run(item_indices, Item):
        idx3 = item_indices.reshape(NW, nchunk, 128)
        return gather_k(idx3, Item)

    return run


def _mlp_body(x_ref, u_ref, w1a_ref, w1b_ref, b1_ref, w2_ref, b2_ref,
              w3_ref, b3_ref, wo_ref, bo_ref, out_ref):
    f32 = jnp.float32
    u = u_ref[...]
    h0 = jnp.dot(u, w1a_ref[...], preferred_element_type=f32) + b1_ref[...]
    x = x_ref[...]
    h1 = jnp.maximum(jnp.dot(x, w1b_ref[...], preferred_element_type=f32) + h0, 0.0)
    h2 = jnp.maximum(jnp.dot(h1, w2_ref[...], preferred_element_type=f32) + b2_ref[...], 0.0)
    h3 = jnp.maximum(jnp.dot(h2, w3_ref[...], preferred_element_type=f32) + b3_ref[...], 0.0)
    logit = jnp.dot(h3, wo_ref[...], preferred_element_type=f32) + bo_ref[...]
    out_ref[...] = jax.nn.sigmoid(logit)


def _mlp(x, u, W1a, W1b, b1, W2, b2, W3, b3, Wo, bo, blk=2048, interpret=False):
    B, D = x.shape
    H1 = W1b.shape[1]
    H2 = W2.shape[1]
    H3 = W3.shape[1]
    grid = (B // blk,)
    full = lambda shape: pl.BlockSpec(shape, lambda i: (0, 0))
    return pl.pallas_call(
        _mlp_body,
        grid=grid,
        in_specs=[
            pl.BlockSpec((blk, D), lambda i: (i, 0)),
            full((1, D)),
            full((D, H1)),
            full((D, H1)),
            full((1, H1)),
            full((H1, H2)),
            full((1, H2)),
            full((H2, H3)),
            full((1, H3)),
            full((H3, 1)),
            full((1, 1)),
        ],
        out_specs=pl.BlockSpec((blk, 1), lambda i: (i, 0)),
        out_shape=jax.ShapeDtypeStruct((B, 1), jnp.float32),
        compiler_params=pltpu.CompilerParams(
            dimension_semantics=("parallel",),
        ),
        interpret=interpret,
    )(x, u, W1a, W1b, b1, W2, b2, W3, b3, Wo, bo)


def kernel(item_indices, Pu, Eu, Item, W1, b1, W2, b2, W3, b3, Wo, bo):
    B = item_indices.shape[0]
    V, D = Item.shape
    gather = _make_sc_gather(V, D, B)
    x = gather(item_indices.astype(jnp.int32), Item)
    u = (Pu + Eu).reshape(1, D)
    W1a = W1[:D, :]
    W1b = W1[D:, :]
    return _mlp(
        x, u, W1a, W1b,
        b1.reshape(1, -1), W2, b2.reshape(1, -1),
        W3, b3.reshape(1, -1), Wo, bo.reshape(1, 1),
    )
